# Initial kernel scaffold; baseline (speedup 1.0000x reference)
import jax
import jax.numpy as jnp
from jax.experimental import pallas as pl


def kernel(atom_features, deg_slice, membership):
    raise NotImplementedError("write your pallas kernel here")


# speedup vs baseline: 3.5451x; 3.5451x over previous
import jax
import jax.numpy as jnp
from jax import lax
from jax.experimental import pallas as pl
from jax.experimental.pallas import tpu as pltpu
from jax.experimental.pallas import tpu_sc as plsc

N = 640000
D = 128
B = 10000

NW = 16
P = N // NW
C = 320
NCH = P // C
OUT_ROWS = NW * 640
INIT_CH = 32
NF = D // 16

_F32 = jnp.float32
_I32 = jnp.int32


def _sc_body(af_hbm, mem_hbm, out_sum, out_max,
             fbuf, mbuf, initbuf, rs, rm, part, pids,
             sp_part, sp_ids, compart, cidv):
    sid = lax.axis_index("s")
    base = sid * P
    neg = jnp.full((16,), -jnp.inf, _F32)
    zero = jnp.zeros((16,), _F32)
    lane = lax.iota(_I32, 16)

    rows_per_tile = OUT_ROWS // NW

    for i in range(INIT_CH):
        for f in range(NF):
            initbuf[pl.ds(i * D + f * 16, 16)] = neg

    def _initmax(j, _):
        off = pl.multiple_of((sid * rows_per_tile + j * INIT_CH) * D, 128)
        pltpu.sync_copy(initbuf, out_max.at[pl.ds(off, INIT_CH * D)])
        return 0
    lax.fori_loop(0, rows_per_tile // INIT_CH, _initmax, 0)

    for i in range(INIT_CH):
        for f in range(NF):
            initbuf[pl.ds(i * D + f * 16, 16)] = zero

    def _initsum(j, _):
        off = pl.multiple_of((sid * rows_per_tile + j * INIT_CH) * D, 128)
        pltpu.sync_copy(initbuf, out_sum.at[pl.ds(off, INIT_CH * D)])
        return 0
    lax.fori_loop(0, rows_per_tile // INIT_CH, _initsum, 0)

    plsc.subcore_barrier()

    pltpu.sync_copy(mem_hbm.at[pl.ds(base, 16)], mbuf.at[pl.ds(0, 16)])
    m0 = mbuf[pl.ds(0, 16)][0]

    for f in range(NF):
        part[pl.ds(f * 16, 16)] = zero
        part[pl.ds(D + f * 16, 16)] = neg

    def _chunk(c, carry):
        row0 = base + c * C
        pltpu.sync_copy(af_hbm.at[pl.ds(row0 * D, C * D)], fbuf)
        pltpu.sync_copy(mem_hbm.at[pl.ds(row0, C)], mbuf)

        def _group(g, carry):
            mv = mbuf[pl.ds(pl.multiple_of(g * 16, 16), 16)]

            for r in range(16):
                cur, fdone, acc = carry
                m = mv[r]
                flag = m != cur
                j = g * 16 + r
                rv = [fbuf[pl.ds(pl.multiple_of(j * D + f * 16, 16), 16)]
                      for f in range(NF)]

                def _flush(cur=cur, fdone=fdone, acc=acc):
                    for f in range(NF):
                        rs[pl.ds(f * 16, 16)] = acc[f]
                        rm[pl.ds(f * 16, 16)] = acc[NF + f]

                    def _interior():
                        off = pl.multiple_of(cur * D, 128)
                        pltpu.sync_copy(rs, out_sum.at[pl.ds(off, D)])
                        pltpu.sync_copy(rm, out_max.at[pl.ds(off, D)])

                    def _first():
                        for f in range(NF):
                            part[pl.ds(f * 16, 16)] = acc[f]
                            part[pl.ds(D + f * 16, 16)] = acc[NF + f]

                    lax.cond(fdone == 1, _interior, _first)

                lax.cond(flag, _flush, lambda: None)
                cur = jnp.where(flag, m, cur)
                fdone = jnp.where(flag, jnp.int32(1), fdone)
                acc = ([jnp.where(flag, zero, acc[f]) + rv[f]
                        for f in range(NF)] +
                       [jnp.maximum(jnp.where(flag, neg, acc[NF + f]), rv[f])
                        for f in range(NF)])
                carry = (cur, fdone, acc)
            return carry

        return lax.fori_loop(0, C // 16, _group, carry)

    carry = (m0, jnp.int32(0), [zero] * NF + [neg] * NF)
    cur, fdone, acc = lax.fori_loop(0, NCH, _chunk, carry)

    for f in range(NF):
        part[pl.ds(2 * D + f * 16, 16)] = acc[f]
        part[pl.ds(3 * D + f * 16, 16)] = acc[NF + f]
    pids[...] = jnp.where(lane == 0,
                          jnp.zeros((16,), _I32) + m0,
                          jnp.zeros((16,), _I32) + cur)

    pltpu.sync_copy(part, sp_part.at[pl.ds(sid * 4 * D, 4 * D)])
    pltpu.sync_copy(pids, sp_ids.at[pl.ds(sid * 16, 16)])

    plsc.subcore_barrier()

    @pl.when(sid == 0)
    def _():
        pltpu.sync_copy(sp_part, compart)
        pltpu.sync_copy(sp_ids, cidv)

        def _writeout(cur2, a):
            for f in range(NF):
                rs[pl.ds(f * 16, 16)] = a[f]
                rm[pl.ds(f * 16, 16)] = a[NF + f]
            off = pl.multiple_of(cur2 * D, 128)
            pltpu.sync_copy(rs, out_sum.at[pl.ds(off, D)])
            pltpu.sync_copy(rm, out_max.at[pl.ds(off, D)])

        def _merge_step(s, carry):
            iv = cidv[pl.ds(pl.multiple_of(s * 16, 16), 16)]
            for t in range(2):
                cur2, a = carry
                eid = iv[t]
                off = pl.multiple_of((2 * s + t) * 2 * D, 32)
                v = ([compart[pl.ds(off + f * 16, 16)]
                      for f in range(NF)] +
                     [compart[pl.ds(off + D + f * 16, 16)]
                      for f in range(NF)])
                flag = eid != cur2

                def _w(cur2=cur2, a=a):
                    _writeout(cur2, a)
                lax.cond(flag & (cur2 >= 0), _w, lambda: None)
                a_new = ([jnp.where(flag, v[f], a[f] + v[f])
                          for f in range(NF)] +
                         [jnp.where(flag, v[NF + f],
                                    jnp.maximum(a[NF + f], v[NF + f]))
                          for f in range(NF)])
                carry = (jnp.where(flag, eid, cur2), a_new)
            return carry

        cur2, a = lax.fori_loop(
            0, NW, _merge_step,
            (jnp.int32(-1), [zero] * NF + [neg] * NF))
        _writeout(cur2, a)


@jax.jit
def _graph_gather(atom_features, membership):
    mesh = plsc.VectorSubcoreMesh(core_axis_name="c", subcore_axis_name="s",
                                  num_cores=1)
    k = pl.kernel(
        _sc_body,
        out_type=(jax.ShapeDtypeStruct((OUT_ROWS * D,), _F32),
                  jax.ShapeDtypeStruct((OUT_ROWS * D,), _F32)),
        mesh=mesh,
        scratch_types=[
            pltpu.VMEM((C * D,), _F32),
            pltpu.VMEM((C,), _I32),
            pltpu.VMEM((INIT_CH * D,), _F32),
            pltpu.VMEM((D,), _F32),
            pltpu.VMEM((D,), _F32),
            pltpu.VMEM((4 * D,), _F32),
            pltpu.VMEM((16,), _I32),
            pltpu.VMEM_SHARED((NW * 4 * D,), _F32),
            pltpu.VMEM_SHARED((NW * 16,), _I32),
            pltpu.VMEM((NW * 4 * D,), _F32),
            pltpu.VMEM((NW * 16,), _I32),
        ],
    )
    out_sum, out_max = k(atom_features.reshape(N * D), membership)
    return jnp.concatenate([out_sum.reshape(OUT_ROWS, D)[:B],
                            out_max.reshape(OUT_ROWS, D)[:B]], axis=1)


def kernel(atom_features, deg_slice, membership):
    del deg_slice
    return _graph_gather(atom_features, membership.astype(_I32))


# speedup vs baseline: 4.3692x; 1.2325x over previous
import jax
import jax.numpy as jnp
from jax import lax
from jax.experimental import pallas as pl
from jax.experimental.pallas import tpu as pltpu
from jax.experimental.pallas import tpu_sc as plsc

N = 640000
D = 128
B = 10000

NW = 16
P = N // NW
C = 320
NCH = P // C
OUT_ROWS = NW * 640
INIT_CH = 32
NF = D // 16

_F32 = jnp.float32
_I32 = jnp.int32


def _sc_body(af_hbm, mem_hbm, out_sum, out_max,
             fbuf, mbuf, initbuf, rs, rm, part, pids,
             sp_part, sp_ids, compart, cidv, accs):
    sid = lax.axis_index("s")
    base = sid * P
    neg = jnp.full((16,), -jnp.inf, _F32)
    zero = jnp.zeros((16,), _F32)
    lane = lax.iota(_I32, 16)

    rows_per_tile = OUT_ROWS // NW

    for i in range(INIT_CH):
        for f in range(NF):
            initbuf[pl.ds(i * D + f * 16, 16)] = neg

    def _initmax(j, _):
        off = pl.multiple_of((sid * rows_per_tile + j * INIT_CH) * D, 128)
        pltpu.sync_copy(initbuf, out_max.at[pl.ds(off, INIT_CH * D)])
        return 0
    lax.fori_loop(0, rows_per_tile // INIT_CH, _initmax, 0)

    for i in range(INIT_CH):
        for f in range(NF):
            initbuf[pl.ds(i * D + f * 16, 16)] = zero

    def _initsum(j, _):
        off = pl.multiple_of((sid * rows_per_tile + j * INIT_CH) * D, 128)
        pltpu.sync_copy(initbuf, out_sum.at[pl.ds(off, INIT_CH * D)])
        return 0
    lax.fori_loop(0, rows_per_tile // INIT_CH, _initsum, 0)

    plsc.subcore_barrier()

    pltpu.sync_copy(mem_hbm.at[pl.ds(base, 16)], mbuf.at[pl.ds(0, 16)])
    m0 = mbuf[pl.ds(0, 16)][0]

    for f in range(NF):
        part[pl.ds(f * 16, 16)] = zero
        part[pl.ds(D + f * 16, 16)] = neg
        accs[pl.ds(f * 16, 16)] = zero
        accs[pl.ds(D + f * 16, 16)] = neg

    def _chunk(c, carry):
        row0 = base + c * C
        pltpu.sync_copy(af_hbm.at[pl.ds(row0 * D, C * D)], fbuf)
        pltpu.sync_copy(mem_hbm.at[pl.ds(row0, C)], mbuf)

        def _group(g, carry):
            mv = mbuf[pl.ds(pl.multiple_of(g * 16, 16), 16)]
            last = mv[15]

            def _rv(r, f):
                j = g * 16 + r
                return fbuf[pl.ds(pl.multiple_of(j * D + f * 16, 16), 16)]

            def _fast(cur, fdone):
                a = ([accs[pl.ds(f * 16, 16)] for f in range(NF)] +
                     [accs[pl.ds(D + f * 16, 16)] for f in range(NF)])
                for r in range(16):
                    a = ([a[f] + _rv(r, f) for f in range(NF)] +
                         [jnp.maximum(a[NF + f], _rv(r, f))
                          for f in range(NF)])
                for f in range(NF):
                    accs[pl.ds(f * 16, 16)] = a[f]
                    accs[pl.ds(D + f * 16, 16)] = a[NF + f]
                return cur, fdone

            def _slow(cur, fdone):
                a = ([accs[pl.ds(f * 16, 16)] for f in range(NF)] +
                     [accs[pl.ds(D + f * 16, 16)] for f in range(NF)])
                for r in range(16):
                    m = mv[r]
                    flag = m != cur
                    rv = [_rv(r, f) for f in range(NF)]

                    def _flush(cur=cur, fdone=fdone, a=a):
                        for f in range(NF):
                            rs[pl.ds(f * 16, 16)] = a[f]
                            rm[pl.ds(f * 16, 16)] = a[NF + f]

                        def _interior():
                            off = pl.multiple_of(cur * D, 128)
                            pltpu.sync_copy(rs, out_sum.at[pl.ds(off, D)])
                            pltpu.sync_copy(rm, out_max.at[pl.ds(off, D)])

                        def _first():
                            for f in range(NF):
                                part[pl.ds(f * 16, 16)] = a[f]
                                part[pl.ds(D + f * 16, 16)] = a[NF + f]

                        lax.cond(fdone == 1, _interior, _first)

                    lax.cond(flag, _flush, lambda: None)
                    cur = jnp.where(flag, m, cur)
                    fdone = jnp.where(flag, jnp.int32(1), fdone)
                    a = ([jnp.where(flag, zero, a[f]) + rv[f]
                          for f in range(NF)] +
                         [jnp.maximum(jnp.where(flag, neg, a[NF + f]), rv[f])
                          for f in range(NF)])
                for f in range(NF):
                    accs[pl.ds(f * 16, 16)] = a[f]
                    accs[pl.ds(D + f * 16, 16)] = a[NF + f]
                return cur, fdone

            cur, fdone = carry
            return lax.cond(last == cur, _fast, _slow, cur, fdone)

        return lax.fori_loop(0, C // 16, _group, carry)

    cur, fdone = lax.fori_loop(0, NCH, _chunk, (m0, jnp.int32(0)))

    for f in range(NF):
        part[pl.ds(2 * D + f * 16, 16)] = accs[pl.ds(f * 16, 16)]
        part[pl.ds(3 * D + f * 16, 16)] = accs[pl.ds(D + f * 16, 16)]
    pids[...] = jnp.where(lane == 0,
                          jnp.zeros((16,), _I32) + m0,
                          jnp.zeros((16,), _I32) + cur)

    pltpu.sync_copy(part, sp_part.at[pl.ds(sid * 4 * D, 4 * D)])
    pltpu.sync_copy(pids, sp_ids.at[pl.ds(sid * 16, 16)])

    plsc.subcore_barrier()

    @pl.when(sid == 0)
    def _():
        pltpu.sync_copy(sp_part, compart)
        pltpu.sync_copy(sp_ids, cidv)

        def _writeout(cur2, a):
            for f in range(NF):
                rs[pl.ds(f * 16, 16)] = a[f]
                rm[pl.ds(f * 16, 16)] = a[NF + f]
            off = pl.multiple_of(cur2 * D, 128)
            pltpu.sync_copy(rs, out_sum.at[pl.ds(off, D)])
            pltpu.sync_copy(rm, out_max.at[pl.ds(off, D)])

        def _merge_step(s, carry):
            iv = cidv[pl.ds(pl.multiple_of(s * 16, 16), 16)]
            for t in range(2):
                cur2, a = carry
                eid = iv[t]
                off = pl.multiple_of((2 * s + t) * 2 * D, 32)
                v = ([compart[pl.ds(off + f * 16, 16)]
                      for f in range(NF)] +
                     [compart[pl.ds(off + D + f * 16, 16)]
                      for f in range(NF)])
                flag = eid != cur2

                def _w(cur2=cur2, a=a):
                    _writeout(cur2, a)
                lax.cond(flag & (cur2 >= 0), _w, lambda: None)
                a_new = ([jnp.where(flag, v[f], a[f] + v[f])
                          for f in range(NF)] +
                         [jnp.where(flag, v[NF + f],
                                    jnp.maximum(a[NF + f], v[NF + f]))
                          for f in range(NF)])
                carry = (jnp.where(flag, eid, cur2), a_new)
            return carry

        cur2, a = lax.fori_loop(
            0, NW, _merge_step,
            (jnp.int32(-1), [zero] * NF + [neg] * NF))
        _writeout(cur2, a)


@jax.jit
def _graph_gather(atom_features, membership):
    mesh = plsc.VectorSubcoreMesh(core_axis_name="c", subcore_axis_name="s",
                                  num_cores=1)
    k = pl.kernel(
        _sc_body,
        out_type=(jax.ShapeDtypeStruct((OUT_ROWS * D,), _F32),
                  jax.ShapeDtypeStruct((OUT_ROWS * D,), _F32)),
        mesh=mesh,
        scratch_types=[
            pltpu.VMEM((C * D,), _F32),
            pltpu.VMEM((C,), _I32),
            pltpu.VMEM((INIT_CH * D,), _F32),
            pltpu.VMEM((D,), _F32),
            pltpu.VMEM((D,), _F32),
            pltpu.VMEM((4 * D,), _F32),
            pltpu.VMEM((16,), _I32),
            pltpu.VMEM_SHARED((NW * 4 * D,), _F32),
            pltpu.VMEM_SHARED((NW * 16,), _I32),
            pltpu.VMEM((NW * 4 * D,), _F32),
            pltpu.VMEM((NW * 16,), _I32),
            pltpu.VMEM((2 * D,), _F32),
        ],
    )
    out_sum, out_max = k(atom_features.reshape(N * D), membership)
    return jnp.concatenate([out_sum.reshape(OUT_ROWS, D)[:B],
                            out_max.reshape(OUT_ROWS, D)[:B]], axis=1)


def kernel(atom_features, deg_slice, membership):
    del deg_slice
    return _graph_gather(atom_features, membership.astype(_I32))


# speedup vs baseline: 6.8933x; 1.5777x over previous
import jax
import jax.numpy as jnp
from jax import lax
from jax.experimental import pallas as pl
from jax.experimental.pallas import tpu as pltpu
from jax.experimental.pallas import tpu_sc as plsc

N = 640000
D = 128
B = 10000

NW = 16
P = N // NW
C = 400
NCH = P // C
OUT_ROWS = NW * 640
INIT_CH = 32
NF = D // 16

_F32 = jnp.float32
_I32 = jnp.int32


def _sc_body(af_hbm, mem_hbm, out_sum, out_max,
             fbuf, mbuf, initbuf, rs, rm, part, pids,
             sp_part, sp_ids, compart, cidv, accs,
             asem0, asem1, msem0, msem1):
    sid = lax.axis_index("s")
    base = sid * P
    neg = jnp.full((16,), -jnp.inf, _F32)
    zero = jnp.zeros((16,), _F32)
    lane = lax.iota(_I32, 16)
    asem = (asem0, asem1)
    msem = (msem0, msem1)

    def _af_copy(c, p):
        return pltpu.make_async_copy(
            af_hbm.at[pl.ds((base + c * C) * D, C * D)],
            fbuf.at[pl.ds(p * C * D, C * D)], asem[p])

    def _mem_copy(c, p):
        return pltpu.make_async_copy(
            mem_hbm.at[pl.ds(base + c * C, C)],
            mbuf.at[pl.ds(p * C, C)], msem[p])

    def _start(c, p):
        _af_copy(c, p).start()
        _mem_copy(c, p).start()

    def _wait(c, p):
        _af_copy(c, p).wait()
        _mem_copy(c, p).wait()

    _start(0, 0)
    _start(1, 1)

    rows_per_tile = OUT_ROWS // NW

    for i in range(INIT_CH):
        for f in range(NF):
            initbuf[pl.ds(i * D + f * 16, 16)] = neg

    def _initmax(j, _):
        off = pl.multiple_of((sid * rows_per_tile + j * INIT_CH) * D, 128)
        pltpu.sync_copy(initbuf, out_max.at[pl.ds(off, INIT_CH * D)])
        return 0
    lax.fori_loop(0, rows_per_tile // INIT_CH, _initmax, 0)

    for i in range(INIT_CH):
        for f in range(NF):
            initbuf[pl.ds(i * D + f * 16, 16)] = zero

    def _initsum(j, _):
        off = pl.multiple_of((sid * rows_per_tile + j * INIT_CH) * D, 128)
        pltpu.sync_copy(initbuf, out_sum.at[pl.ds(off, INIT_CH * D)])
        return 0
    lax.fori_loop(0, rows_per_tile // INIT_CH, _initsum, 0)

    plsc.subcore_barrier()

    for f in range(NF):
        part[pl.ds(f * 16, 16)] = zero
        part[pl.ds(D + f * 16, 16)] = neg
        accs[pl.ds(f * 16, 16)] = zero
        accs[pl.ds(D + f * 16, 16)] = neg

    def _process(p, carry):
        def _group(g, carry):
            mv = mbuf[pl.ds(pl.multiple_of(p * C + g * 16, 16), 16)]
            last = mv[15]

            def _rv(r, f):
                j = g * 16 + r
                return fbuf[pl.ds(
                    pl.multiple_of(p * C * D + j * D + f * 16, 16), 16)]

            def _fast(cur, fdone):
                a = ([accs[pl.ds(f * 16, 16)] for f in range(NF)] +
                     [accs[pl.ds(D + f * 16, 16)] for f in range(NF)])
                for r in range(16):
                    a = ([a[f] + _rv(r, f) for f in range(NF)] +
                         [jnp.maximum(a[NF + f], _rv(r, f))
                          for f in range(NF)])
                for f in range(NF):
                    accs[pl.ds(f * 16, 16)] = a[f]
                    accs[pl.ds(D + f * 16, 16)] = a[NF + f]
                return cur, fdone

            def _slow(cur, fdone):
                a = ([accs[pl.ds(f * 16, 16)] for f in range(NF)] +
                     [accs[pl.ds(D + f * 16, 16)] for f in range(NF)])
                for r in range(16):
                    m = mv[r]
                    flag = m != cur
                    rv = [_rv(r, f) for f in range(NF)]

                    def _flush(cur=cur, fdone=fdone, a=a):
                        for f in range(NF):
                            rs[pl.ds(f * 16, 16)] = a[f]
                            rm[pl.ds(f * 16, 16)] = a[NF + f]

                        def _interior():
                            off = pl.multiple_of(cur * D, 128)
                            pltpu.sync_copy(rs, out_sum.at[pl.ds(off, D)])
                            pltpu.sync_copy(rm, out_max.at[pl.ds(off, D)])

                        def _first():
                            for f in range(NF):
                                part[pl.ds(f * 16, 16)] = a[f]
                                part[pl.ds(D + f * 16, 16)] = a[NF + f]

                        lax.cond(fdone == 1, _interior, _first)

                    lax.cond(flag, _flush, lambda: None)
                    cur = jnp.where(flag, m, cur)
                    fdone = jnp.where(flag, jnp.int32(1), fdone)
                    a = ([jnp.where(flag, zero, a[f]) + rv[f]
                          for f in range(NF)] +
                         [jnp.maximum(jnp.where(flag, neg, a[NF + f]), rv[f])
                          for f in range(NF)])
                for f in range(NF):
                    accs[pl.ds(f * 16, 16)] = a[f]
                    accs[pl.ds(D + f * 16, 16)] = a[NF + f]
                return cur, fdone

            cur, fdone = carry
            return lax.cond(last == cur, _fast, _slow, cur, fdone)

        return lax.fori_loop(0, C // 16, _group, carry)

    _wait(0, 0)
    m0 = mbuf[pl.ds(0, 16)][0]

    def _pair(cc, carry):
        c0 = 2 * cc
        carry = _process(0, carry)
        lax.cond(c0 + 2 < NCH, lambda: _start(c0 + 2, 0), lambda: None)
        _wait(c0 + 1, 1)
        carry = _process(1, carry)
        lax.cond(c0 + 3 < NCH, lambda: _start(c0 + 3, 1), lambda: None)
        lax.cond(c0 + 2 < NCH, lambda: _wait(c0 + 2, 0), lambda: None)
        return carry

    cur, fdone = lax.fori_loop(0, NCH // 2, _pair, (m0, jnp.int32(0)))

    for f in range(NF):
        part[pl.ds(2 * D + f * 16, 16)] = accs[pl.ds(f * 16, 16)]
        part[pl.ds(3 * D + f * 16, 16)] = accs[pl.ds(D + f * 16, 16)]
    pids[...] = jnp.where(lane == 0,
                          jnp.zeros((16,), _I32) + m0,
                          jnp.zeros((16,), _I32) + cur)

    pltpu.sync_copy(part, sp_part.at[pl.ds(sid * 4 * D, 4 * D)])
    pltpu.sync_copy(pids, sp_ids.at[pl.ds(sid * 16, 16)])

    plsc.subcore_barrier()

    @pl.when(sid == 0)
    def _():
        pltpu.sync_copy(sp_part, compart)
        pltpu.sync_copy(sp_ids, cidv)

        def _writeout(cur2, a):
            for f in range(NF):
                rs[pl.ds(f * 16, 16)] = a[f]
                rm[pl.ds(f * 16, 16)] = a[NF + f]
            off = pl.multiple_of(cur2 * D, 128)
            pltpu.sync_copy(rs, out_sum.at[pl.ds(off, D)])
            pltpu.sync_copy(rm, out_max.at[pl.ds(off, D)])

        def _merge_step(s, carry):
            iv = cidv[pl.ds(pl.multiple_of(s * 16, 16), 16)]
            for t in range(2):
                cur2, a = carry
                eid = iv[t]
                off = pl.multiple_of((2 * s + t) * 2 * D, 32)
                v = ([compart[pl.ds(off + f * 16, 16)]
                      for f in range(NF)] +
                     [compart[pl.ds(off + D + f * 16, 16)]
                      for f in range(NF)])
                flag = eid != cur2

                def _w(cur2=cur2, a=a):
                    _writeout(cur2, a)
                lax.cond(flag & (cur2 >= 0), _w, lambda: None)
                a_new = ([jnp.where(flag, v[f], a[f] + v[f])
                          for f in range(NF)] +
                         [jnp.where(flag, v[NF + f],
                                    jnp.maximum(a[NF + f], v[NF + f]))
                          for f in range(NF)])
                carry = (jnp.where(flag, eid, cur2), a_new)
            return carry

        cur2, a = lax.fori_loop(
            0, NW, _merge_step,
            (jnp.int32(-1), [zero] * NF + [neg] * NF))
        _writeout(cur2, a)


@jax.jit
def _graph_gather(atom_features, membership):
    mesh = plsc.VectorSubcoreMesh(core_axis_name="c", subcore_axis_name="s",
                                  num_cores=1)
    k = pl.kernel(
        _sc_body,
        out_type=(jax.ShapeDtypeStruct((OUT_ROWS * D,), _F32),
                  jax.ShapeDtypeStruct((OUT_ROWS * D,), _F32)),
        mesh=mesh,
        scratch_types=[
            pltpu.VMEM((2 * C * D,), _F32),
            pltpu.VMEM((2 * C,), _I32),
            pltpu.VMEM((INIT_CH * D,), _F32),
            pltpu.VMEM((D,), _F32),
            pltpu.VMEM((D,), _F32),
            pltpu.VMEM((4 * D,), _F32),
            pltpu.VMEM((16,), _I32),
            pltpu.VMEM_SHARED((NW * 4 * D,), _F32),
            pltpu.VMEM_SHARED((NW * 16,), _I32),
            pltpu.VMEM((NW * 4 * D,), _F32),
            pltpu.VMEM((NW * 16,), _I32),
            pltpu.VMEM((2 * D,), _F32),
            pltpu.SemaphoreType.DMA,
            pltpu.SemaphoreType.DMA,
            pltpu.SemaphoreType.DMA,
            pltpu.SemaphoreType.DMA,
        ],
    )
    out_sum, out_max = k(atom_features.reshape(N * D), membership)
    return jnp.concatenate([out_sum.reshape(OUT_ROWS, D)[:B],
                            out_max.reshape(OUT_ROWS, D)[:B]], axis=1)


def kernel(atom_features, deg_slice, membership):
    del deg_slice
    return _graph_gather(atom_features, membership.astype(_I32))
